# baseline (speedup 1.0000x reference)
import jax
import jax.numpy as jnp
from jax.experimental import pallas as pl
from jax.experimental.pallas import tpu as pltpu

N, T, FEAT, C, U, V = 4, 512, 80, 256, 48, 256
BLANK = 0
UP = 56
TB = 128
NEG = -1e30
D_TOT = T + U + 1


def _phase1(x_ref, we_ref, be_ref, oh_ref, ohs_ref, emb_ref, wj_ref, bj_ref,
            lpb_ref, lps_ref, dec_ref):
    xb = x_ref[0]
    enc = jnp.dot(xb, we_ref[...], preferred_element_type=jnp.float32) + be_ref[0]

    @pl.when(pl.program_id(1) == 0)
    def _():
        oh = oh_ref[0]
        dec_ref[...] = jnp.dot(oh, emb_ref[...],
                               preferred_element_type=jnp.float32)

    dec = dec_ref[...]
    joint = jnp.tanh(enc.astype(jnp.bfloat16)[:, None, :]
                     + dec.astype(jnp.bfloat16)[None, :, :])
    logits = jnp.dot(joint.reshape(TB * UP, C), wj_ref[...],
                     preferred_element_type=jnp.float32) + bj_ref[0]
    l3 = logits.reshape(TB, UP, V)
    lse = jnp.log(jnp.sum(jnp.exp(l3), axis=2))
    iota_v = jax.lax.broadcasted_iota(jnp.int32, (TB, UP, V), 2)
    lpb = jnp.sum(jnp.where(iota_v == BLANK, l3, 0.0), axis=2) - lse
    sym = jnp.sum(l3 * ohs_ref[0][None], axis=2) - lse
    ucol = jax.lax.broadcasted_iota(jnp.int32, (TB, UP), 1)
    lpb_ref[0] = jnp.where(ucol <= U, lpb, NEG)
    lps_ref[0] = jnp.where(ucol < U, sym, NEG)


def _shu(v, k=1):
    return jnp.concatenate(
        [jnp.full(v.shape[:-1] + (k,), NEG, jnp.float32), v[..., :-k]],
        axis=-1)


def _lse2(p, q):
    m = jnp.maximum(p, q)
    return m + jnp.log1p(jnp.exp(jnp.minimum(p, q) - m))


def _lse3(p, q, r):
    m = jnp.maximum(jnp.maximum(p, q), r)
    return m + jnp.log(jnp.exp(p - m) + jnp.exp(q - m) + jnp.exp(r - m))


def _phase2(pk_ref, out_ref, w2_ref, w4_ref):
    u_iota = jax.lax.broadcasted_iota(jnp.int32, (N, UP), 1)
    a_init = jnp.where(u_iota == 0, 0.0, NEG)
    out_ref[0] = a_init

    allr = pk_ref[...]
    sbo, sbe = allr[:, 0:N], allr[:, N:2 * N]
    sso, sse = allr[:, 2 * N:3 * N], allr[:, 3 * N:4 * N]
    w0 = sbo + sbe
    w1 = jnp.logaddexp(sso + sbe, _shu(sbo) + sse)
    w2 = _shu(sso) + sse
    w2_ref[...] = jnp.concatenate([w0, w1, w2], axis=1)

    def split(v):
        vv = v.reshape(140, 2, N, UP)
        return vv[:, 1], vv[:, 0]

    a0w, b0w = split(w0)
    a1w, b1w = split(w1)
    a2w, b2w = split(w2)
    w4_0 = a0w + b0w
    w4_1 = jnp.logaddexp(a0w + b1w, a1w + _shu(b0w))
    w4_2 = _lse3(a0w + b2w, a1w + _shu(b1w), a2w + _shu(b0w, 2))
    w4_3 = jnp.logaddexp(a1w + _shu(b2w), a2w + _shu(b1w, 2))
    w4_4 = a2w + _shu(b2w, 2)
    w4_ref[...] = jnp.concatenate([w4_0, w4_1, w4_2, w4_3, w4_4],
                                  axis=1)

    def body(i, a):
        pk_a = pk_ref[2 * i]
        pk_b = pk_ref[2 * i + 1]
        w2m = w2_ref[2 * i]
        w4 = w4_ref[i]
        a1 = _shu(a)
        a2 = _shu(a1)
        a3 = _shu(a2)
        a4 = _shu(a3)
        s1 = _lse2(a + pk_a[0:N], a1 + pk_a[2 * N:3 * N])
        out_ref[4 * i + 1] = s1
        s2 = _lse3(a + w2m[0:N], a1 + w2m[N:2 * N], a2 + w2m[2 * N:3 * N])
        out_ref[4 * i + 2] = s2
        s3 = _lse2(s2 + pk_b[0:N], _shu(s2) + pk_b[2 * N:3 * N])
        out_ref[4 * i + 3] = s3
        t0 = a + w4[0:N]
        t1 = a1 + w4[N:2 * N]
        t2 = a2 + w4[2 * N:3 * N]
        t3 = a3 + w4[3 * N:4 * N]
        t4 = a4 + w4[4 * N:5 * N]
        m = jnp.maximum(jnp.maximum(jnp.maximum(t0, t1), jnp.maximum(t2, t3)),
                        t4)
        s = (jnp.exp(t0 - m) + jnp.exp(t1 - m) + jnp.exp(t2 - m)
             + jnp.exp(t3 - m) + jnp.exp(t4 - m))
        a_new = m + jnp.log(s)
        out_ref[4 * i + 4] = a_new
        return a_new

    jax.lax.fori_loop(0, (D_TOT - 1) // 4, body, a_init, unroll=2)


def _shear(arrT, left_pad, width):
    w = width + left_pad
    p = jnp.pad(arrT, ((0, 0), (0, 0), (left_pad, (w + UP + 1) - w)),
                constant_values=NEG)
    flat = p.reshape(N, UP * (w + UP + 1))[:, :UP * (w + UP)]
    sh = flat.reshape(N, UP, w + UP)[:, :, :D_TOT]
    return jnp.transpose(sh, (2, 0, 1))


def kernel(x, x_lens, y_padded, y_lens, W_enc, b_enc, emb, W_join, b_join):
    f32 = jnp.float32
    sos_y = jnp.concatenate(
        [jnp.zeros((N, 1), y_padded.dtype), y_padded], axis=1)
    sos_pad = jnp.pad(sos_y, ((0, 0), (0, UP - (U + 1))))
    vio = jnp.arange(V, dtype=sos_pad.dtype)
    oh = (sos_pad[:, :, None] == vio).astype(f32)
    yp_pad = jnp.pad(y_padded, ((0, 0), (0, UP - U)), constant_values=-1)
    ohs = (yp_pad[:, :, None] == vio).astype(f32)

    lpb, lps = pl.pallas_call(
        _phase1,
        grid=(N, T // TB),
        in_specs=[
            pl.BlockSpec((1, TB, FEAT), lambda n, t: (n, t, 0)),
            pl.BlockSpec((FEAT, C), lambda n, t: (0, 0)),
            pl.BlockSpec((1, C), lambda n, t: (0, 0)),
            pl.BlockSpec((1, UP, V), lambda n, t: (n, 0, 0)),
            pl.BlockSpec((1, UP, V), lambda n, t: (n, 0, 0)),
            pl.BlockSpec((V, C), lambda n, t: (0, 0)),
            pl.BlockSpec((C, V), lambda n, t: (0, 0)),
            pl.BlockSpec((1, V), lambda n, t: (0, 0)),
        ],
        out_specs=[
            pl.BlockSpec((1, TB, UP), lambda n, t: (n, t, 0)),
            pl.BlockSpec((1, TB, UP), lambda n, t: (n, t, 0)),
        ],
        out_shape=[
            jax.ShapeDtypeStruct((N, T, UP), f32),
            jax.ShapeDtypeStruct((N, T, UP), f32),
        ],
        scratch_shapes=[pltpu.VMEM((UP, C), jnp.float32)],
    )(x.astype(jnp.bfloat16), W_enc.astype(jnp.bfloat16),
      b_enc.reshape(1, C).astype(f32), oh, ohs, emb.astype(f32),
      W_join.astype(jnp.bfloat16), b_join.reshape(1, V).astype(f32))

    sb = _shear(jnp.transpose(lpb, (0, 2, 1)), 1, T)
    lpsT = jnp.transpose(lps, (0, 2, 1))
    ls2 = jnp.pad(lpsT, ((0, 0), (1, 0), (0, 0)),
                  constant_values=NEG)[:, :UP]
    ss = _shear(ls2, 0, T)

    dh = (D_TOT - 1) // 2
    packed = jnp.concatenate(
        [sb[1:].reshape(dh, 2 * N, UP), ss[1:].reshape(dh, 2 * N, UP)],
        axis=1)

    alphas = pl.pallas_call(
        _phase2,
        out_shape=jax.ShapeDtypeStruct((D_TOT, N, UP), f32),
        scratch_shapes=[pltpu.VMEM((dh, 3 * N, UP), jnp.float32),
                        pltpu.VMEM((dh // 2, 5 * N, UP), jnp.float32)],
    )(packed)

    n_idx = jnp.arange(N)
    dn = x_lens - 1 + y_lens
    final_alpha = alphas[dn, n_idx, y_lens]
    final_blank = lpb[n_idx, x_lens - 1, y_lens]
    return -jnp.sum(final_alpha + final_blank)
